# baseline (speedup 1.0000x reference)
import functools

import jax
import jax.numpy as jnp
from jax import lax
from jax.experimental import pallas as pl
from jax.experimental.pallas import tpu as pltpu
from jax.experimental.pallas import tpu_sc as plsc

VOCAB = 64
HIDDEN = 1024
LANES = 16
NUM_CORES = 2
NUM_SUBCORES = 16
NW = NUM_CORES * NUM_SUBCORES
CHUNK = 32
NBUF = 3


def _build_pair_table(table):
    rows_per_step = 8

    def body(row_ref, full_ref, out_ref):
        left = jnp.broadcast_to(
            row_ref[...][:, None, :], (rows_per_step, VOCAB, HIDDEN)
        )
        right = jnp.broadcast_to(
            full_ref[...][None], (rows_per_step, VOCAB, HIDDEN)
        )
        out_ref[...] = jnp.concatenate([left, right], axis=-1)

    out = pl.pallas_call(
        body,
        grid=(VOCAB // rows_per_step,),
        in_specs=[
            pl.BlockSpec((rows_per_step, HIDDEN), lambda i: (i, 0)),
            pl.BlockSpec((VOCAB, HIDDEN), lambda i: (0, 0)),
        ],
        out_specs=pl.BlockSpec(
            (rows_per_step, VOCAB, 2 * HIDDEN), lambda i: (i, 0, 0)
        ),
        out_shape=jax.ShapeDtypeStruct((VOCAB, VOCAB, 2 * HIDDEN), jnp.float32),
    )(table, table)
    return out.reshape(VOCAB * VOCAB, 2 * HIDDEN)


def _make_kernel(batch_tokens):
    b_per_w = batch_tokens // NW
    pairs_per_w = b_per_w // 2
    n_chunks = b_per_w // CHUNK
    pair_chunk = CHUNK // 2
    mesh = plsc.VectorSubcoreMesh(core_axis_name="c", subcore_axis_name="s")

    @functools.partial(
        pl.kernel,
        mesh=mesh,
        compiler_params=pltpu.CompilerParams(needs_layout_passes=False),
        out_type=jax.ShapeDtypeStruct((batch_tokens // 2, 2 * HIDDEN), jnp.float32),
        scratch_types=[
            pltpu.VMEM((pairs_per_w,), jnp.int32),
            pltpu.VMEM((pairs_per_w,), jnp.int32),
            pltpu.VMEM((n_chunks, pair_chunk), jnp.int32),
            pltpu.VMEM((b_per_w,), jnp.float32),
            pltpu.VMEM((NBUF, pair_chunk, 2 * HIDDEN), jnp.float32),
            pltpu.SemaphoreType.DMA((NBUF,)),
            pltpu.SemaphoreType.DMA((NBUF,)),
        ],
    )
    def k(ids_even_hbm, ids_odd_hbm, mask_hbm, table_hbm, out_hbm,
          ev_v, od_v, idx_v, mask_v, rows_v, sem_g, sem_w):
        wid = lax.axis_index("s") * NUM_CORES + lax.axis_index("c")
        base_p = wid * pairs_per_w
        pltpu.sync_copy(ids_even_hbm.at[wid], ev_v)
        pltpu.sync_copy(ids_odd_hbm.at[wid], od_v)
        pltpu.sync_copy(mask_hbm.at[wid], mask_v)

        for r in range(n_chunks):
            for v in range(pair_chunk // LANES):
                sl = pl.ds(r * pair_chunk + v * LANES, LANES)
                pi = ev_v[sl] * VOCAB + od_v[sl]
                idx_v[r, pl.ds(v * LANES, LANES)] = pi

        def start_gather(c):
            return pltpu.async_copy(
                table_hbm.at[idx_v.at[c]], rows_v.at[c % NBUF], sem_g.at[c % NBUF]
            )

        def start_write(c):
            return pltpu.async_copy(
                rows_v.at[c % NBUF],
                out_hbm.at[pl.ds(base_p + c * pair_chunk, pair_chunk)],
                sem_w.at[c % NBUF],
            )

        gathers = {0: start_gather(0)}
        writes = {}
        for c in range(n_chunks):
            b = c % NBUF
            if c >= 2:
                writes.pop(c - 2).wait()
            if c + 1 < n_chunks:
                gathers[c + 1] = start_gather(c + 1)
            gathers.pop(c).wait()

            def scale_token(t, _):
                m = plsc.load_gather(
                    mask_v, [jnp.full((LANES,), c * CHUNK + t, jnp.int32)]
                )
                p = t // 2
                off = (t % 2) * HIDDEN
                for k16 in range(HIDDEN // LANES):
                    sl = pl.ds(off + k16 * LANES, LANES)
                    rows_v[b, p, sl] = rows_v[b, p, sl] * m
                return 0

            lax.fori_loop(0, CHUNK, scale_token, 0)
            writes[c] = start_write(c)
        writes.pop(n_chunks - 2).wait()
        writes.pop(n_chunks - 1).wait()

    return k


def kernel(input_ids, attention_mask, word_embeddings):
    batch, seq = input_ids.shape
    tokens = batch * seq
    ids2 = input_ids.reshape(NW, tokens // NW // 2, 2).astype(jnp.int32)
    ids_even = ids2[:, :, 0]
    ids_odd = ids2[:, :, 1]
    mask = attention_mask.reshape(NW, tokens // NW).astype(jnp.float32)
    pair_table = _build_pair_table(word_embeddings)
    out = _make_kernel(tokens)(ids_even, ids_odd, mask, pair_table)
    return out.reshape(batch, seq, HIDDEN)


# speedup vs baseline: 2.0669x; 2.0669x over previous
import functools

import jax
import jax.numpy as jnp
from jax import lax
from jax.experimental import pallas as pl
from jax.experimental.pallas import tpu as pltpu
from jax.experimental.pallas import tpu_sc as plsc

VOCAB = 64
HIDDEN = 1024
LANES = 16
NUM_CORES = 2
NUM_SUBCORES = 16
NW = NUM_CORES * NUM_SUBCORES
CHUNK = 32
NBUF = 3


def _replicate_table(table):
    def body(t_ref, out_ref):
        out_ref[...] = jnp.broadcast_to(t_ref[...], (NW, VOCAB, HIDDEN))

    return pl.pallas_call(
        body,
        out_shape=jax.ShapeDtypeStruct((NW, VOCAB, HIDDEN), jnp.float32),
    )(table)


def _make_kernel(batch_tokens):
    b_per_w = batch_tokens // NW
    n_chunks = b_per_w // CHUNK
    mesh = plsc.VectorSubcoreMesh(core_axis_name="c", subcore_axis_name="s")

    @functools.partial(
        pl.kernel,
        mesh=mesh,
        compiler_params=pltpu.CompilerParams(needs_layout_passes=False),
        out_type=jax.ShapeDtypeStruct((batch_tokens, HIDDEN), jnp.float32),
        scratch_types=[
            pltpu.VMEM((n_chunks, CHUNK), jnp.int32),
            pltpu.VMEM((b_per_w,), jnp.float32),
            pltpu.VMEM((NBUF, CHUNK, HIDDEN), jnp.float32),
            pltpu.SemaphoreType.DMA((NBUF,)),
            pltpu.SemaphoreType.DMA((NBUF,)),
        ],
    )
    def k(ids_hbm, mask_hbm, table_hbm, out_hbm, idx_v, mask_v, rows_v,
          sem_g, sem_w):
        wid = lax.axis_index("s") * NUM_CORES + lax.axis_index("c")
        base = wid * b_per_w
        pltpu.sync_copy(ids_hbm.at[wid], idx_v)
        pltpu.sync_copy(mask_hbm.at[wid], mask_v)
        my_table = table_hbm.at[wid]

        def start_gather(c):
            return pltpu.async_copy(
                my_table.at[idx_v.at[c]], rows_v.at[c % NBUF], sem_g.at[c % NBUF]
            )

        def start_write(c):
            return pltpu.async_copy(
                rows_v.at[c % NBUF],
                out_hbm.at[pl.ds(base + c * CHUNK, CHUNK)],
                sem_w.at[c % NBUF],
            )

        gathers = {0: start_gather(0)}
        writes = {}
        for c in range(n_chunks):
            b = c % NBUF
            if c >= 2:
                writes.pop(c - 2).wait()
            if c + 1 < n_chunks:
                gathers[c + 1] = start_gather(c + 1)
            gathers.pop(c).wait()

            def scale_token(t, _):
                m = plsc.load_gather(
                    mask_v, [jnp.full((LANES,), c * CHUNK + t, jnp.int32)]
                )
                for k16 in range(HIDDEN // LANES):
                    sl = pl.ds(k16 * LANES, LANES)
                    rows_v[b, t, sl] = rows_v[b, t, sl] * m
                return 0

            lax.fori_loop(0, CHUNK, scale_token, 0)
            writes[c] = start_write(c)
        writes.pop(n_chunks - 2).wait()
        writes.pop(n_chunks - 1).wait()

    return k


def kernel(input_ids, attention_mask, word_embeddings):
    batch, seq = input_ids.shape
    tokens = batch * seq
    ids = input_ids.reshape(NW, tokens // NW // CHUNK, CHUNK).astype(jnp.int32)
    mask = attention_mask.reshape(NW, tokens // NW).astype(jnp.float32)
    table_rep = _replicate_table(word_embeddings)
    out = _make_kernel(tokens)(ids, mask, table_rep)
    return out.reshape(batch, seq, HIDDEN)
